# baseline (speedup 1.0000x reference)
import jax
import jax.numpy as jnp
from jax import lax
from jax.experimental import pallas as pl
from jax.experimental.pallas import tpu as pltpu
from jax.experimental.pallas import tpu_sc as plsc

N_NODES = 10000
C = 128
CH = C // 2
N_EDGES = 320000

NC = 2
NS = 16
K = 128
CHUNKS = 2 * (-(-N_EDGES // (NS * K * 2)))
PER_S = CHUNKS * K
E_PAD = PER_S * NS
NP = N_NODES // 2
SLABP = 312
TAILP = NP - NS * SLABP


def _sc_body(xP, meta, wgt, out, xsp, acc, midx, wbuf, rows, msg,
             gsem0, gsem1, msem0, msem1, wsem0, wsem1):
    cid = lax.axis_index("c")
    sid = lax.axis_index("s")
    gsems = (gsem0, gsem1)
    msems = (msem0, msem1)
    wsems = (wsem0, wsem1)

    def meta_cp(ch, b):
        return pltpu.make_async_copy(meta.at[sid, ch], midx.at[b], msems[b])

    def wgt_cp(ch, b):
        return pltpu.make_async_copy(wgt.at[sid, ch], wbuf.at[b], wsems[b])

    def gather(ch, b):
        return pltpu.make_async_copy(
            xsp.at[midx.at[b, 1]], rows.at[b], gsems[b])

    p0 = pl.multiple_of(sid * SLABP, 8)
    poff = 0
    while poff < SLABP:
        n = min(K, SLABP - poff)
        pltpu.sync_copy(xP.at[cid, pl.ds(p0 + poff, n)],
                        xsp.at[pl.ds(p0 + poff, n)])
        poff += n

    @pl.when(sid == 0)
    def _stage_tail():
        pltpu.sync_copy(xP.at[cid, pl.ds(NS * SLABP, TAILP)],
                        xsp.at[pl.ds(NS * SLABP, TAILP)])

    def zero_row(i, _):
        for j in range(C // 16):
            msg[i, pl.ds(16 * j, 16)] = jnp.zeros((16,), jnp.float32)
        return 0
    lax.fori_loop(0, K, zero_row, 0)
    poff = 0
    while poff < SLABP:
        n = min(K, SLABP - poff)
        pltpu.sync_copy(msg.at[pl.ds(0, n)], acc.at[pl.ds(p0 + poff, n)])
        poff += n

    @pl.when(sid == 0)
    def _zero_tail():
        pltpu.sync_copy(msg.at[pl.ds(0, TAILP)], acc.at[pl.ds(NS * SLABP, TAILP)])
    plsc.subcore_barrier()

    meta_cp(0, 0).start()
    wgt_cp(0, 0).start()
    meta_cp(1, 1).start()
    wgt_cp(1, 1).start()
    meta_cp(0, 0).wait()
    wgt_cp(0, 0).wait()
    gather(0, 0).start()

    zeros16 = jnp.zeros((16,), jnp.float32)

    def pair(g, _):
        for b in range(2):
            t = 2 * g + b
            b1 = 1 - b
            gather(t, b).wait()

            @pl.when(t + 1 < CHUNKS)
            def _next_gather():
                meta_cp(t + 1, b1).wait()
                wgt_cp(t + 1, b1).wait()
                gather(t + 1, b1).start()

            def scale(g8, _):
                wv = wbuf[b, pl.ds(16 * g8, 16)]
                w2v = wv * wv
                iov = midx[b, 2, pl.ds(16 * g8, 16)]
                jov = midx[b, 3, pl.ds(16 * g8, 16)]
                for l in range(16):
                    e = 16 * g8 + l
                    w2 = w2v[l]
                    io = iov[l]
                    jo = jov[l]
                    oio = 64 - io
                    for k in range(CH // 16):
                        msg[e, pl.ds(io + 16 * k, 16)] = (
                            rows[b, e, pl.ds(jo + 16 * k, 16)] * w2)
                        msg[e, pl.ds(oio + 16 * k, 16)] = zeros16
                return 0
            lax.fori_loop(0, K // 16, scale, 0)

            pltpu.sync_copy(msg, acc.at[midx.at[b, 0]], add=True)

            @pl.when(t + 2 < CHUNKS)
            def _prefetch_meta():
                meta_cp(t + 2, b).start()
                wgt_cp(t + 2, b).start()
        return 0
    lax.fori_loop(0, CHUNKS // 2, pair, 0)

    plsc.subcore_barrier()
    poff = 0
    while poff < SLABP:
        n = min(K, SLABP - poff)
        pltpu.sync_copy(acc.at[pl.ds(p0 + poff, n)],
                        out.at[cid, pl.ds(p0 + poff, n)])
        poff += n

    @pl.when(sid == 0)
    def _write_tail():
        pltpu.sync_copy(acc.at[pl.ds(NS * SLABP, TAILP)],
                        out.at[cid, pl.ds(NS * SLABP, TAILP)])


def _combine_body(p_ref, o_ref):
    o_ref[0] = jnp.concatenate([p_ref[0].T, p_ref[1].T], axis=0)


_combine = pl.pallas_call(
    _combine_body,
    out_shape=jax.ShapeDtypeStruct((1, C, N_NODES), jnp.float32),
)


def kernel(x, iInd, jInd, W):
    xT = jnp.swapaxes(x[0], 0, 1)
    xP = jnp.stack([xT[:, :CH].reshape(NP, C), xT[:, CH:].reshape(NP, C)])
    pad = E_PAD - iInd.shape[0]
    iP = jnp.concatenate([iInd, jnp.zeros((pad,), jnp.int32)])
    jP = jnp.concatenate([jInd, jnp.zeros((pad,), jnp.int32)])
    wP = jnp.concatenate([W, jnp.zeros((pad,), jnp.float32)])
    meta = jnp.concatenate([
        (iP >> 1).reshape(NS, CHUNKS, 1, K),
        (jP >> 1).reshape(NS, CHUNKS, 1, K),
        (CH * (iP & 1)).reshape(NS, CHUNKS, 1, K),
        (CH * (jP & 1)).reshape(NS, CHUNKS, 1, K),
    ], axis=2)
    wgt = wP.reshape(NS, CHUNKS, K)

    sc = pl.kernel(
        _sc_body,
        out_type=jax.ShapeDtypeStruct((NC, NP, C), jnp.float32),
        mesh=plsc.VectorSubcoreMesh(core_axis_name="c", subcore_axis_name="s"),
        scratch_types=[
            pltpu.VMEM_SHARED((NP, C), jnp.float32),
            pltpu.VMEM_SHARED((NP, C), jnp.float32),
            pltpu.VMEM((2, 4, K), jnp.int32),
            pltpu.VMEM((2, K), jnp.float32),
            pltpu.VMEM((2, K, C), jnp.float32),
            pltpu.VMEM((K, C), jnp.float32),
            pltpu.SemaphoreType.DMA,
            pltpu.SemaphoreType.DMA,
            pltpu.SemaphoreType.DMA,
            pltpu.SemaphoreType.DMA,
            pltpu.SemaphoreType.DMA,
            pltpu.SemaphoreType.DMA,
        ],
    )
    partial = sc(xP, meta, wgt)
    ph = partial.reshape(NC, N_NODES, CH)
    return _combine(ph)
